# baseline (speedup 1.0000x reference)
import functools

import jax
import jax.numpy as jnp
from jax import lax
from jax.experimental import pallas as pl
from jax.experimental.pallas import tpu as pltpu
from jax.experimental.pallas import tpu_sc as plsc

NUM_FEATURES = 128
NUM_EDGES = 160000
NUM_NODES = 10000
ROWS = NUM_NODES + NUM_EDGES
BATCH = 4
TOTAL_ROWS = BATCH * ROWS

ZCHUNK = 17000
N_Z = TOTAL_ROWS // ZCHUNK

NC, NS = 2, 16
NW = NC * NS
N_IDX = 128
ROWS_PER_W = BATCH * N_IDX // NW
IDX_GROUPS = N_IDX // ROWS_PER_W


def _fill_body(out_ref, zbuf, sem_z):
    zbuf[...] = jnp.zeros_like(zbuf)
    for k in range(N_Z):
        pltpu.make_async_copy(
            zbuf, out_ref.at[pl.ds(k * ZCHUNK, ZCHUNK)], sem_z
        ).start()
    for k in range(N_Z):
        pltpu.make_async_copy(
            zbuf, out_ref.at[pl.ds(k * ZCHUNK, ZCHUNK)], sem_z
        ).wait()


def _tc_fill():
    return pl.pallas_call(
        _fill_body,
        grid=(1,),
        out_specs=pl.BlockSpec(memory_space=pl.ANY),
        out_shape=jax.ShapeDtypeStruct((TOTAL_ROWS, NUM_FEATURES), jnp.float32),
        scratch_shapes=[
            pltpu.VMEM((ZCHUNK, NUM_FEATURES), jnp.float32),
            pltpu.SemaphoreType.DMA,
        ],
    )()


_sc_mesh = plsc.VectorSubcoreMesh(core_axis_name="c", subcore_axis_name="s")


@functools.partial(
    pl.kernel,
    out_type=(),
    mesh=_sc_mesh,
    scratch_types=[
        pltpu.VMEM((ROWS_PER_W,), jnp.int32),
        pltpu.VMEM((ROWS_PER_W, NUM_FEATURES), jnp.float32),
        pltpu.SemaphoreType.DMA,
        pltpu.SemaphoreType.DMA,
    ],
)
def _sc_scatter(out_ref, x_hbm, idx_hbm, idx_v, rows_v, sem_i, sem_x):
    wid = lax.axis_index("s") * NC + lax.axis_index("c")
    b = wid // IDX_GROUPS
    g = wid % IDX_GROUPS
    cp_i = pltpu.async_copy(idx_hbm.at[g], idx_v, sem_i)
    cp_x = pltpu.async_copy(x_hbm.at[pl.ds(wid * ROWS_PER_W, ROWS_PER_W)], rows_v, sem_x)
    cp_i.wait()
    idx_v[...] = idx_v[...] + (b * ROWS + NUM_EDGES)
    cp_x.wait()
    pltpu.sync_copy(rows_v, out_ref.at[idx_v])


def kernel(x, pivotal_nodes):
    bsz, n_idx, f = x.shape
    x_flat = x.reshape(bsz * n_idx, f)
    idx2 = pivotal_nodes.reshape(IDX_GROUPS, ROWS_PER_W)
    out_ref = jax.new_ref(_tc_fill())
    _sc_scatter(out_ref, x_flat, idx2)
    return out_ref[...].reshape(bsz, ROWS, f)
